# baseline (speedup 1.0000x reference)
import functools

import jax
import jax.numpy as jnp
from jax import lax
from jax.experimental import pallas as pl
from jax.experimental.pallas import tpu as pltpu
from jax.experimental.pallas import tpu_sc as plsc

NW = 32
CHUNK = 128
KD = 5
KA = 4
OC = 64


def _sc_mesh():
    return plsc.VectorSubcoreMesh(core_axis_name="c", subcore_axis_name="s")


def _make_deg_kernel(N_PAD, stripe, chunks):
    batches = chunks // KD

    @functools.partial(
        pl.kernel,
        out_type=jax.ShapeDtypeStruct((2 * N_PAD,), jnp.float32),
        mesh=_sc_mesh(),
        scratch_types=[
            pltpu.VMEM((chunks, CHUNK), jnp.int32),
            pltpu.VMEM((CHUNK,), jnp.float32),
            pltpu.VMEM((stripe,), jnp.float32),
            [pltpu.SemaphoreType.DMA] * KD,
            pltpu.VMEM_SHARED((N_PAD,), jnp.float32),
        ],
    )
    def deg_kernel(eidx_hbm, out_hbm, idx2d, onesv, stagev, sems, acc_sh):
        c = lax.axis_index("c")
        s = lax.axis_index("s")
        wid = s * 2 + c
        pltpu.sync_copy(eidx_hbm.at[wid], idx2d)
        for i in range(CHUNK // 16):
            onesv[pl.ds(i * 16, 16)] = jnp.ones((16,), jnp.float32)

        def zfill(j, carry):
            stagev[pl.ds(j * 16, 16)] = jnp.zeros((16,), jnp.float32)
            return carry

        lax.fori_loop(0, stripe // 16, zfill, 0)
        pltpu.sync_copy(stagev, acc_sh.at[pl.ds(s * stripe, stripe)])
        plsc.subcore_barrier()

        def body(b, carry):
            cps = []
            for i in range(KD):
                cps.append(pltpu.async_copy(
                    onesv, acc_sh.at[idx2d.at[b * KD + i]], sems[i], add=True))
            for cp in cps:
                cp.wait()
            return carry

        lax.fori_loop(0, batches, body, 0)
        plsc.subcore_barrier()
        pltpu.sync_copy(acc_sh.at[pl.ds(s * stripe, stripe)], stagev)
        pltpu.sync_copy(stagev,
                        out_hbm.at[pl.ds(c * N_PAD + s * stripe, stripe)])

    return deg_kernel


def _make_agg_kernel(NL_PAD, D, stripe, chunks):
    batches = chunks // KA

    @functools.partial(
        pl.kernel,
        out_type=jax.ShapeDtypeStruct((2, NL_PAD, D), jnp.float32),
        mesh=_sc_mesh(),
        scratch_types=[
            pltpu.VMEM((chunks, CHUNK), jnp.int32),
            pltpu.VMEM((chunks, CHUNK), jnp.int32),
            [pltpu.VMEM((CHUNK, D), jnp.float32)] * KA,
            [pltpu.SemaphoreType.DMA] * KA,
            [pltpu.SemaphoreType.DMA] * KA,
            pltpu.VMEM_SHARED((NL_PAD, D), jnp.float32),
        ],
    )
    def agg_kernel(eidx_hbm, col_hbm, y_hbm, out_hbm,
                   idx2d, col2d, bufs, gsems, ssems, acc_sh):
        c = lax.axis_index("c")
        s = lax.axis_index("s")
        wid = s * 2 + c
        pltpu.sync_copy(eidx_hbm.at[wid], idx2d)
        pltpu.sync_copy(col_hbm.at[wid], col2d)

        def zrow(i, carry):
            for k in range(D // 16):
                bufs[0][i, pl.ds(k * 16, 16)] = jnp.zeros((16,), jnp.float32)
            return carry

        lax.fori_loop(0, OC, zrow, 0)

        def zcp(t, carry):
            pltpu.sync_copy(bufs[0].at[pl.ds(0, OC), :],
                            acc_sh.at[pl.ds(s * stripe + t * OC, OC), :])
            return carry

        lax.fori_loop(0, stripe // OC, zcp, 0)

        for i in range(KA):
            pltpu.async_copy(y_hbm.at[col2d.at[i]], bufs[i], gsems[i])
        plsc.subcore_barrier()

        def body(b, carry):
            cps = []
            for i in range(KA):
                cidx = b * KA + i
                pltpu.make_async_copy(
                    y_hbm.at[col2d.at[cidx]], bufs[i], gsems[i]).wait()
                cps.append(pltpu.async_copy(
                    bufs[i], acc_sh.at[idx2d.at[cidx]], ssems[i], add=True))
            for i in range(KA):
                cidx = b * KA + i
                cps[i].wait()

                @pl.when(b < batches - 1)
                def _(i=i, cidx=cidx):
                    pltpu.async_copy(
                        y_hbm.at[col2d.at[cidx + KA]], bufs[i], gsems[i])
            return carry

        lax.fori_loop(0, batches, body, 0)
        plsc.subcore_barrier()

        def ocp(t, carry):
            base2 = s * stripe + t * OC
            pltpu.sync_copy(acc_sh.at[pl.ds(base2, OC), :],
                            bufs[0].at[pl.ds(0, OC), :])
            pltpu.sync_copy(bufs[0].at[pl.ds(0, OC), :],
                            out_hbm.at[c, pl.ds(base2, OC), :])
            return carry

        lax.fori_loop(0, stripe // OC, ocp, 0)

    return agg_kernel


def _make_eidx_body(half, dummy_g, dummy_l):
    def eidx_body(row_ref, col_ref, eg_ref, ea_ref, eb_ref):
        r = row_ref[...]
        self_loop = r == col_ref[...]
        eg_ref[...] = jnp.where(self_loop, dummy_g, r)
        ea_ref[...] = jnp.where(self_loop | (r >= half), dummy_l, r)
        eb_ref[...] = jnp.where(self_loop | (r < half), dummy_l, r - half)
    return eidx_body


def _y_body(x_ref, degp_ref, y_ref):
    d = degp_ref[:, 0] + degp_ref[:, 1] + 1.0
    dis = lax.rsqrt(d)
    y_ref[...] = x_ref[...] * dis[:, None]


def _m_body(accp_ref, y_ref, degp_ref, w_ref, b_ref, h_ref, stats_ref):
    i = pl.program_id(0)
    d = degp_ref[:, 0] + degp_ref[:, 1] + 1.0
    dis = lax.rsqrt(d)
    pre = (accp_ref[0] + accp_ref[1] + y_ref[...]) * dis[:, None]
    h = lax.dot_general(pre, w_ref[...], (((1,), (1,)), ((), ())),
                        preferred_element_type=jnp.float32)
    h = jnp.maximum(h + b_ref[...], 0.0)
    h_ref[...] = h

    @pl.when(i == 0)
    def _():
        stats_ref[...] = jnp.zeros_like(stats_ref)

    stats_ref[0:1, :] += jnp.sum(h, axis=0, keepdims=True)
    stats_ref[1:2, :] += jnp.sum(h * h, axis=0, keepdims=True)


def _make_bn_body(N):
    def bn_body(h_ref, stats_ref, gamma_ref, beta_ref, out_ref):
        inv_n = 1.0 / N
        mean = stats_ref[0:1, :] * inv_n
        ex2 = stats_ref[1:2, :] * inv_n
        var = ex2 - mean * mean
        inv = lax.rsqrt(var + 1e-5)
        out_ref[...] = (h_ref[...] - mean) * inv * gamma_ref[...] + beta_ref[...]
    return bn_body


def kernel(x, edge_index, W, b, gamma, beta):
    N, D = x.shape
    E = edge_index.shape[1]
    N_PAD = ((N + 8) + 255) // 256 * 256
    stripe = N_PAD // 16
    HALF = N // 2
    NL_PAD = (HALF + 8 + 255) // 256 * 256
    stripe_l = NL_PAD // 16
    DUMMY_G = N_PAD - 8
    DUMMY_L = NL_PAD - 8
    align = NW * CHUNK * KD * KA
    E_PAD = (E + align - 1) // align * align

    row = edge_index[0]
    col = edge_index[1]

    eidx_g, eidx_a, eidx_b = pl.pallas_call(
        _make_eidx_body(HALF, DUMMY_G, DUMMY_L),
        in_specs=[pl.BlockSpec((E // CHUNK, CHUNK), lambda: (0, 0))] * 2,
        out_specs=[pl.BlockSpec((E // CHUNK, CHUNK), lambda: (0, 0))] * 3,
        out_shape=[jax.ShapeDtypeStruct((E // CHUNK, CHUNK), jnp.int32)] * 3,
    )(row.reshape(E // CHUNK, CHUNK), col.reshape(E // CHUNK, CHUNK))

    npad = E_PAD - E
    eidx_g = jnp.concatenate(
        [eidx_g.reshape(E), jnp.full((npad,), DUMMY_G, jnp.int32)])
    eidx_a = jnp.concatenate(
        [eidx_a.reshape(E), jnp.full((npad,), DUMMY_L, jnp.int32)])
    eidx_b = jnp.concatenate(
        [eidx_b.reshape(E), jnp.full((npad,), DUMMY_L, jnp.int32)])
    col_p = jnp.concatenate([col, jnp.zeros((npad,), jnp.int32)])

    chunks = E_PAD // (NW * CHUNK)

    deg_p = _make_deg_kernel(N_PAD, stripe, chunks)(
        eidx_g.reshape(NW, chunks, CHUNK))
    degp_t = deg_p.reshape(2, N_PAD).T

    BN = 1000
    grid = (N // BN,)
    y = pl.pallas_call(
        _y_body,
        grid=grid,
        in_specs=[
            pl.BlockSpec((BN, D), lambda i: (i, 0)),
            pl.BlockSpec((BN, 2), lambda i: (i, 0)),
        ],
        out_specs=pl.BlockSpec((BN, D), lambda i: (i, 0)),
        out_shape=jax.ShapeDtypeStruct((N, D), jnp.float32),
    )(x, degp_t)

    agg = _make_agg_kernel(NL_PAD, D, stripe_l, chunks)
    col3 = col_p.reshape(NW, chunks, CHUNK)
    acc_a = agg(eidx_a.reshape(NW, chunks, CHUNK), col3, y)
    acc_b = agg(eidx_b.reshape(NW, chunks, CHUNK), col3, y)
    acc_p = jnp.concatenate([acc_a[:, :HALF], acc_b[:, :HALF]], axis=1)

    b2 = b.reshape(1, D)
    h, stats = pl.pallas_call(
        _m_body,
        grid=grid,
        in_specs=[
            pl.BlockSpec((2, BN, D), lambda i: (0, i, 0)),
            pl.BlockSpec((BN, D), lambda i: (i, 0)),
            pl.BlockSpec((BN, 2), lambda i: (i, 0)),
            pl.BlockSpec((D, D), lambda i: (0, 0)),
            pl.BlockSpec((1, D), lambda i: (0, 0)),
        ],
        out_specs=[
            pl.BlockSpec((BN, D), lambda i: (i, 0)),
            pl.BlockSpec((8, D), lambda i: (0, 0)),
        ],
        out_shape=[
            jax.ShapeDtypeStruct((N, D), jnp.float32),
            jax.ShapeDtypeStruct((8, D), jnp.float32),
        ],
    )(acc_p, y, degp_t, W, b2)

    out = pl.pallas_call(
        _make_bn_body(N),
        grid=grid,
        in_specs=[
            pl.BlockSpec((BN, D), lambda i: (i, 0)),
            pl.BlockSpec((8, D), lambda i: (0, 0)),
            pl.BlockSpec((1, D), lambda i: (0, 0)),
            pl.BlockSpec((1, D), lambda i: (0, 0)),
        ],
        out_specs=pl.BlockSpec((BN, D), lambda i: (i, 0)),
        out_shape=jax.ShapeDtypeStruct((N, D), jnp.float32),
    )(h, stats, gamma.reshape(1, D), beta.reshape(1, D))
    return out


# speedup vs baseline: 1.8000x; 1.8000x over previous
import functools

import jax
import jax.numpy as jnp
from jax import lax
from jax.experimental import pallas as pl
from jax.experimental.pallas import tpu as pltpu
from jax.experimental.pallas import tpu_sc as plsc

NW = 32
CHUNK = 128
KD = 5
KA = 2


def _sc_mesh():
    return plsc.VectorSubcoreMesh(core_axis_name="c", subcore_axis_name="s")


def _make_deg_kernel(N_PAD, stripe, chunks):
    batches = chunks // KD

    @functools.partial(
        pl.kernel,
        out_type=jax.ShapeDtypeStruct((2 * N_PAD,), jnp.float32),
        mesh=_sc_mesh(),
        scratch_types=[
            pltpu.VMEM((chunks, CHUNK), jnp.int32),
            pltpu.VMEM((CHUNK,), jnp.float32),
            pltpu.VMEM((stripe,), jnp.float32),
            [pltpu.SemaphoreType.DMA] * KD,
            pltpu.VMEM_SHARED((N_PAD,), jnp.float32),
        ],
    )
    def deg_kernel(eidx_hbm, out_hbm, idx2d, onesv, stagev, sems, acc_sh):
        c = lax.axis_index("c")
        s = lax.axis_index("s")
        wid = s * 2 + c
        pltpu.sync_copy(eidx_hbm.at[wid], idx2d)
        for i in range(CHUNK // 16):
            onesv[pl.ds(i * 16, 16)] = jnp.ones((16,), jnp.float32)

        def zfill(j, carry):
            stagev[pl.ds(j * 16, 16)] = jnp.zeros((16,), jnp.float32)
            return carry

        lax.fori_loop(0, stripe // 16, zfill, 0)
        pltpu.sync_copy(stagev, acc_sh.at[pl.ds(s * stripe, stripe)])
        plsc.subcore_barrier()

        def body(b, carry):
            cps = []
            for i in range(KD):
                cps.append(pltpu.async_copy(
                    onesv, acc_sh.at[idx2d.at[b * KD + i]], sems[i], add=True))
            for cp in cps:
                cp.wait()
            return carry

        lax.fori_loop(0, batches, body, 0)
        plsc.subcore_barrier()
        pltpu.sync_copy(acc_sh.at[pl.ds(s * stripe, stripe)], stagev)
        pltpu.sync_copy(stagev,
                        out_hbm.at[pl.ds(c * N_PAD + s * stripe, stripe)])

    return deg_kernel


def _make_agg_kernel(N_PAD, D, stripe, chunks):
    hchunks = chunks // 2
    batches = hchunks // KA

    @functools.partial(
        pl.kernel,
        out_type=jax.ShapeDtypeStruct((2, N_PAD, D), jnp.float32),
        mesh=_sc_mesh(),
        scratch_types=[
            pltpu.VMEM((hchunks, CHUNK), jnp.int32),
            pltpu.VMEM((hchunks, CHUNK), jnp.int32),
            [pltpu.VMEM((CHUNK, D), jnp.float32)] * KA,
            [pltpu.SemaphoreType.DMA] * KA,
            [pltpu.SemaphoreType.DMA] * KA,
            pltpu.VMEM_SHARED((N_PAD, D), jnp.float32),
        ],
    )
    def agg_kernel(eidx_hbm, col_hbm, y_hbm, out_hbm,
                   idx2d, col2d, bufs, gsems, ssems, acc_sh):
        c = lax.axis_index("c")
        s = lax.axis_index("s")
        wid = s * 2 + c

        def zrow(i, carry):
            for k in range(D // 16):
                bufs[0][i, pl.ds(k * 16, 16)] = jnp.zeros((16,), jnp.float32)
            return carry

        lax.fori_loop(0, CHUNK, zrow, 0)

        def zcp(t, carry):
            pltpu.sync_copy(bufs[0],
                            acc_sh.at[pl.ds(s * stripe + t * CHUNK, CHUNK), :])
            return carry

        lax.fori_loop(0, stripe // CHUNK, zcp, 0)
        plsc.subcore_barrier()

        for hp in range(2):
            pltpu.sync_copy(
                eidx_hbm.at[wid, pl.ds(hp * hchunks, hchunks), :], idx2d)
            pltpu.sync_copy(
                col_hbm.at[wid, pl.ds(hp * hchunks, hchunks), :], col2d)

            def body(b, carry):
                scps = []
                for i in range(KA):
                    cidx = b * KA + i
                    pltpu.async_copy(
                        y_hbm.at[col2d.at[cidx]], bufs[i], gsems[i]).wait()
                    scps.append(pltpu.async_copy(
                        bufs[i], acc_sh.at[idx2d.at[cidx]], ssems[i],
                        add=True))
                for cp in scps:
                    cp.wait()
                return carry

            lax.fori_loop(0, batches, body, 0)
        plsc.subcore_barrier()

        def ocp(t, carry):
            base2 = s * stripe + t * CHUNK
            pltpu.sync_copy(acc_sh.at[pl.ds(base2, CHUNK), :], bufs[0])
            pltpu.sync_copy(bufs[0], out_hbm.at[c, pl.ds(base2, CHUNK), :])
            return carry

        lax.fori_loop(0, stripe // CHUNK, ocp, 0)

    return agg_kernel


def _make_eidx_body(n, n_dummy):
    def eidx_body(row_ref, col_ref, eg_ref):
        r = row_ref[...]
        eg_ref[...] = jnp.where(r == col_ref[...], n + lax.rem(r, n_dummy), r)
    return eidx_body


def _y_body(x_ref, degp_ref, y_ref):
    d = degp_ref[:, 0] + degp_ref[:, 1] + 1.0
    dis = lax.rsqrt(d)
    y_ref[...] = x_ref[...] * dis[:, None]


def _m_body(accp_ref, y_ref, degp_ref, w_ref, b_ref, h_ref, stats_ref):
    i = pl.program_id(0)
    d = degp_ref[:, 0] + degp_ref[:, 1] + 1.0
    dis = lax.rsqrt(d)
    pre = (accp_ref[0] + accp_ref[1] + y_ref[...]) * dis[:, None]
    h = lax.dot_general(pre, w_ref[...], (((1,), (1,)), ((), ())),
                        preferred_element_type=jnp.float32)
    h = jnp.maximum(h + b_ref[...], 0.0)
    h_ref[...] = h

    @pl.when(i == 0)
    def _():
        stats_ref[...] = jnp.zeros_like(stats_ref)

    stats_ref[0:1, :] += jnp.sum(h, axis=0, keepdims=True)
    stats_ref[1:2, :] += jnp.sum(h * h, axis=0, keepdims=True)


def _make_bn_body(N):
    def bn_body(h_ref, stats_ref, gamma_ref, beta_ref, out_ref):
        inv_n = 1.0 / N
        mean = stats_ref[0:1, :] * inv_n
        ex2 = stats_ref[1:2, :] * inv_n
        var = ex2 - mean * mean
        inv = lax.rsqrt(var + 1e-5)
        out_ref[...] = (h_ref[...] - mean) * inv * gamma_ref[...] + beta_ref[...]
    return bn_body


def kernel(x, edge_index, W, b, gamma, beta):
    N, D = x.shape
    E = edge_index.shape[1]
    N_PAD = ((N + 8) + 255) // 256 * 256
    stripe = N_PAD // 16
    DUMMY_G = N_PAD - 8
    align = NW * CHUNK * 2 * KA
    E_PAD = (E + align - 1) // align * align

    row = edge_index[0]
    col = edge_index[1]

    n_dummy = N_PAD - N - 8
    eidx_g = pl.pallas_call(
        _make_eidx_body(N, n_dummy),
        in_specs=[pl.BlockSpec((E // CHUNK, CHUNK), lambda: (0, 0))] * 2,
        out_specs=pl.BlockSpec((E // CHUNK, CHUNK), lambda: (0, 0)),
        out_shape=jax.ShapeDtypeStruct((E // CHUNK, CHUNK), jnp.int32),
    )(row.reshape(E // CHUNK, CHUNK), col.reshape(E // CHUNK, CHUNK))

    npad = E_PAD - E
    eidx_g = jnp.concatenate(
        [eidx_g.reshape(E),
         N + jnp.arange(npad, dtype=jnp.int32) % n_dummy])
    col_p = jnp.concatenate([col, jnp.zeros((npad,), jnp.int32)])

    chunks = E_PAD // (NW * CHUNK)

    deg_p = _make_deg_kernel(N_PAD, stripe, chunks)(
        eidx_g.reshape(NW, chunks, CHUNK))
    degp_t = deg_p.reshape(2, N_PAD).T

    BN = 1000
    grid = (N // BN,)
    y = pl.pallas_call(
        _y_body,
        grid=grid,
        in_specs=[
            pl.BlockSpec((BN, D), lambda i: (i, 0)),
            pl.BlockSpec((BN, 2), lambda i: (i, 0)),
        ],
        out_specs=pl.BlockSpec((BN, D), lambda i: (i, 0)),
        out_shape=jax.ShapeDtypeStruct((N, D), jnp.float32),
    )(x, degp_t)

    acc_p = _make_agg_kernel(N_PAD, D, stripe, chunks)(
        eidx_g.reshape(NW, chunks, CHUNK),
        col_p.reshape(NW, chunks, CHUNK), y)

    b2 = b.reshape(1, D)
    h, stats = pl.pallas_call(
        _m_body,
        grid=grid,
        in_specs=[
            pl.BlockSpec((2, BN, D), lambda i: (0, i, 0)),
            pl.BlockSpec((BN, D), lambda i: (i, 0)),
            pl.BlockSpec((BN, 2), lambda i: (i, 0)),
            pl.BlockSpec((D, D), lambda i: (0, 0)),
            pl.BlockSpec((1, D), lambda i: (0, 0)),
        ],
        out_specs=[
            pl.BlockSpec((BN, D), lambda i: (i, 0)),
            pl.BlockSpec((8, D), lambda i: (0, 0)),
        ],
        out_shape=[
            jax.ShapeDtypeStruct((N, D), jnp.float32),
            jax.ShapeDtypeStruct((8, D), jnp.float32),
        ],
    )(acc_p, y, degp_t, W, b2)

    out = pl.pallas_call(
        _make_bn_body(N),
        grid=grid,
        in_specs=[
            pl.BlockSpec((BN, D), lambda i: (i, 0)),
            pl.BlockSpec((8, D), lambda i: (0, 0)),
            pl.BlockSpec((1, D), lambda i: (0, 0)),
            pl.BlockSpec((1, D), lambda i: (0, 0)),
        ],
        out_specs=pl.BlockSpec((BN, D), lambda i: (i, 0)),
        out_shape=jax.ShapeDtypeStruct((N, D), jnp.float32),
    )(h, stats, gamma.reshape(1, D), beta.reshape(1, D))
    return out


# speedup vs baseline: 1.9750x; 1.0972x over previous
import functools

import jax
import jax.numpy as jnp
from jax import lax
from jax.experimental import pallas as pl
from jax.experimental.pallas import tpu as pltpu
from jax.experimental.pallas import tpu_sc as plsc

NW = 32
NT = 16
CHUNK = 128
KD = 5
KA = 2


def _sc_mesh():
    return plsc.VectorSubcoreMesh(core_axis_name="c", subcore_axis_name="s")


def _make_deg_kernel(N_PAD, stripe, chunks):
    batches = chunks // KD

    @functools.partial(
        pl.kernel,
        out_type=jax.ShapeDtypeStruct((2 * N_PAD,), jnp.float32),
        mesh=_sc_mesh(),
        scratch_types=[
            pltpu.VMEM((chunks, CHUNK), jnp.int32),
            pltpu.VMEM((CHUNK,), jnp.float32),
            pltpu.VMEM((stripe,), jnp.float32),
            [pltpu.SemaphoreType.DMA] * KD,
            pltpu.VMEM_SHARED((N_PAD,), jnp.float32),
        ],
    )
    def deg_kernel(eidx_hbm, out_hbm, idx2d, onesv, stagev, sems, acc_sh):
        c = lax.axis_index("c")
        s = lax.axis_index("s")
        wid = s * 2 + c
        pltpu.sync_copy(eidx_hbm.at[wid], idx2d)
        for i in range(CHUNK // 16):
            onesv[pl.ds(i * 16, 16)] = jnp.ones((16,), jnp.float32)

        def zfill(j, carry):
            stagev[pl.ds(j * 16, 16)] = jnp.zeros((16,), jnp.float32)
            return carry

        lax.fori_loop(0, stripe // 16, zfill, 0)
        pltpu.sync_copy(stagev, acc_sh.at[pl.ds(s * stripe, stripe)])
        plsc.subcore_barrier()

        def body(b, carry):
            cps = []
            for i in range(KD):
                cps.append(pltpu.async_copy(
                    onesv, acc_sh.at[idx2d.at[b * KD + i]], sems[i], add=True))
            for cp in cps:
                cp.wait()
            return carry

        lax.fori_loop(0, batches, body, 0)
        plsc.subcore_barrier()
        pltpu.sync_copy(acc_sh.at[pl.ds(s * stripe, stripe)], stagev)
        pltpu.sync_copy(stagev,
                        out_hbm.at[pl.ds(c * N_PAD + s * stripe, stripe)])

    return deg_kernel


def _make_agg_kernel(N_PAD, D, stripe, n0, n1, npass):
    batches = npass // KA

    @functools.partial(
        pl.kernel,
        out_type=jax.ShapeDtypeStruct((2, N_PAD, D), jnp.float32),
        mesh=_sc_mesh(),
        scratch_types=[
            pltpu.VMEM((npass, CHUNK), jnp.int32),
            pltpu.VMEM((npass, CHUNK), jnp.int32),
            [pltpu.VMEM((CHUNK, D), jnp.float32)] * KA,
            [pltpu.SemaphoreType.DMA] * KA,
            [pltpu.SemaphoreType.DMA] * KA,
            pltpu.VMEM_SHARED((N_PAD, D), jnp.float32),
        ],
    )
    def agg_kernel(eidx_hbm, col_hbm, y_hbm, out_hbm,
                   idx2d, col2d, bufs, gsems, ssems, acc_sh):
        c = lax.axis_index("c")
        s = lax.axis_index("s")

        def zrow(i, carry):
            for k in range(D // 16):
                bufs[0][i, pl.ds(k * 16, 16)] = jnp.zeros((16,), jnp.float32)
            return carry

        lax.fori_loop(0, CHUNK, zrow, 0)

        def zcp(t, carry):
            pltpu.sync_copy(bufs[0],
                            acc_sh.at[pl.ds(s * stripe + t * CHUNK, CHUNK), :])
            return carry

        lax.fori_loop(0, stripe // CHUNK, zcp, 0)
        plsc.subcore_barrier()

        def run_range(row0):
            pltpu.sync_copy(eidx_hbm.at[pl.ds(row0, npass), :], idx2d)
            pltpu.sync_copy(col_hbm.at[pl.ds(row0, npass), :], col2d)

            def body(b, carry):
                scps = []
                for i in range(KA):
                    cidx = b * KA + i
                    pltpu.async_copy(
                        y_hbm.at[col2d.at[cidx]], bufs[i], gsems[i]).wait()
                    scps.append(pltpu.async_copy(
                        bufs[i], acc_sh.at[idx2d.at[cidx]], ssems[i],
                        add=True))
                for cp in scps:
                    cp.wait()
                return carry

            lax.fori_loop(0, batches, body, 0)

        @pl.when(c == 0)
        def _():
            for hp in range(n0 // npass):
                run_range(s * n0 + hp * npass)

        @pl.when(c == 1)
        def _():
            for hp in range(n1 // npass):
                run_range(16 * n0 + s * n1 + hp * npass)

        plsc.subcore_barrier()

        def ocp(t, carry):
            base2 = s * stripe + t * CHUNK
            pltpu.sync_copy(acc_sh.at[pl.ds(base2, CHUNK), :], bufs[0])
            pltpu.sync_copy(bufs[0], out_hbm.at[c, pl.ds(base2, CHUNK), :])
            return carry

        lax.fori_loop(0, stripe // CHUNK, ocp, 0)

    return agg_kernel


def _make_eidx_body(n, n_dummy):
    def eidx_body(row_ref, col_ref, eg_ref):
        r = row_ref[...]
        eg_ref[...] = jnp.where(r == col_ref[...], n + lax.rem(r, n_dummy), r)
    return eidx_body


def _y_body(x_ref, degp_ref, y_ref):
    d = degp_ref[:, 0] + degp_ref[:, 1] + 1.0
    dis = lax.rsqrt(d)
    y_ref[...] = x_ref[...] * dis[:, None]


def _m_body(accp_ref, y_ref, degp_ref, w_ref, b_ref, h_ref, stats_ref):
    i = pl.program_id(0)
    d = degp_ref[:, 0] + degp_ref[:, 1] + 1.0
    dis = lax.rsqrt(d)
    pre = (accp_ref[0] + accp_ref[1] + y_ref[...]) * dis[:, None]
    h = lax.dot_general(pre, w_ref[...], (((1,), (1,)), ((), ())),
                        preferred_element_type=jnp.float32)
    h = jnp.maximum(h + b_ref[...], 0.0)
    h_ref[...] = h

    @pl.when(i == 0)
    def _():
        stats_ref[...] = jnp.zeros_like(stats_ref)

    stats_ref[0:1, :] += jnp.sum(h, axis=0, keepdims=True)
    stats_ref[1:2, :] += jnp.sum(h * h, axis=0, keepdims=True)


def _make_bn_body(N):
    def bn_body(h_ref, stats_ref, gamma_ref, beta_ref, out_ref):
        inv_n = 1.0 / N
        mean = stats_ref[0:1, :] * inv_n
        ex2 = stats_ref[1:2, :] * inv_n
        var = ex2 - mean * mean
        inv = lax.rsqrt(var + 1e-5)
        out_ref[...] = (h_ref[...] - mean) * inv * gamma_ref[...] + beta_ref[...]
    return bn_body


def kernel(x, edge_index, W, b, gamma, beta):
    N, D = x.shape
    E = edge_index.shape[1]
    N_PAD = ((N + 8) + 255) // 256 * 256
    stripe = N_PAD // 16
    DUMMY_G = N_PAD - 8
    align = NW * CHUNK * 2 * KA
    E_PAD = (E + align - 1) // align * align

    row = edge_index[0]
    col = edge_index[1]

    n_dummy = N_PAD - N - 8
    eidx_g = pl.pallas_call(
        _make_eidx_body(N, n_dummy),
        in_specs=[pl.BlockSpec((E // CHUNK, CHUNK), lambda: (0, 0))] * 2,
        out_specs=pl.BlockSpec((E // CHUNK, CHUNK), lambda: (0, 0)),
        out_shape=jax.ShapeDtypeStruct((E // CHUNK, CHUNK), jnp.int32),
    )(row.reshape(E // CHUNK, CHUNK), col.reshape(E // CHUNK, CHUNK))

    npad = E_PAD - E
    eidx_g = jnp.concatenate(
        [eidx_g.reshape(E),
         N + jnp.arange(npad, dtype=jnp.int32) % n_dummy])
    col_p = jnp.concatenate([col, jnp.zeros((npad,), jnp.int32)])

    chunks = E_PAD // (NW * CHUNK)

    deg_p = _make_deg_kernel(N_PAD, stripe, chunks)(
        eidx_g.reshape(NW, chunks, CHUNK))
    degp_t = deg_p.reshape(2, N_PAD).T

    BN = 1000
    grid = (N // BN,)
    y = pl.pallas_call(
        _y_body,
        grid=grid,
        in_specs=[
            pl.BlockSpec((BN, D), lambda i: (i, 0)),
            pl.BlockSpec((BN, 2), lambda i: (i, 0)),
        ],
        out_specs=pl.BlockSpec((BN, D), lambda i: (i, 0)),
        out_shape=jax.ShapeDtypeStruct((N, D), jnp.float32),
    )(x, degp_t)

    rows_total = E_PAD // CHUNK
    n0 = rows_total * 3 // (4 * NT)
    n1 = rows_total // NT - n0
    acc_p = _make_agg_kernel(N_PAD, D, stripe, n0, n1, n1)(
        eidx_g.reshape(rows_total, CHUNK),
        col_p.reshape(rows_total, CHUNK), y)

    b2 = b.reshape(1, D)
    h, stats = pl.pallas_call(
        _m_body,
        grid=grid,
        in_specs=[
            pl.BlockSpec((2, BN, D), lambda i: (0, i, 0)),
            pl.BlockSpec((BN, D), lambda i: (i, 0)),
            pl.BlockSpec((BN, 2), lambda i: (i, 0)),
            pl.BlockSpec((D, D), lambda i: (0, 0)),
            pl.BlockSpec((1, D), lambda i: (0, 0)),
        ],
        out_specs=[
            pl.BlockSpec((BN, D), lambda i: (i, 0)),
            pl.BlockSpec((8, D), lambda i: (0, 0)),
        ],
        out_shape=[
            jax.ShapeDtypeStruct((N, D), jnp.float32),
            jax.ShapeDtypeStruct((8, D), jnp.float32),
        ],
    )(acc_p, y, degp_t, W, b2)

    out = pl.pallas_call(
        _make_bn_body(N),
        grid=grid,
        in_specs=[
            pl.BlockSpec((BN, D), lambda i: (i, 0)),
            pl.BlockSpec((8, D), lambda i: (0, 0)),
            pl.BlockSpec((1, D), lambda i: (0, 0)),
            pl.BlockSpec((1, D), lambda i: (0, 0)),
        ],
        out_specs=pl.BlockSpec((BN, D), lambda i: (i, 0)),
        out_shape=jax.ShapeDtypeStruct((N, D), jnp.float32),
    )(h, stats, gamma.reshape(1, D), beta.reshape(1, D))
    return out
